# baseline (speedup 1.0000x reference)
import functools

import jax
import jax.numpy as jnp
from jax import lax
from jax.experimental import pallas as pl
from jax.experimental.pallas import tpu as pltpu
from jax.experimental.pallas import tpu_sc as plsc

_CL = 65536


def _proj_body(w_ref, b_ref, x_ref, o_ref):
    x = x_ref[...]
    w = w_ref[...]
    p = jnp.sum(x * w, axis=0, keepdims=True) + b_ref[0, 0]
    o_ref[...] = p.reshape(o_ref.shape)


def _project_table_tc(table_t, w_col, b):
    D, V = table_t.shape
    G = (V + _CL - 1) // _CL
    return pl.pallas_call(
        _proj_body,
        grid=(G,),
        in_specs=[
            pl.BlockSpec((D, 1), lambda i: (0, 0)),
            pl.BlockSpec((1, 1), lambda i: (0, 0)),
            pl.BlockSpec((D, _CL), lambda i: (0, i)),
        ],
        out_specs=pl.BlockSpec((1, 1, _CL), lambda i: (i, 0, 0)),
        out_shape=jax.ShapeDtypeStruct((G, 1, _CL), jnp.float32),
    )(w_col, b.reshape(1, 1), table_t)


def _gather_scalars(proj_flat, ids_flat):
    info = plsc.get_sparse_core_info()
    nc, ns = info.num_cores, info.num_subcores
    nw = nc * ns
    B = ids_flat.shape[0]
    assert B % nw == 0
    bpw = B // nw

    mesh = plsc.VectorSubcoreMesh(core_axis_name="c", subcore_axis_name="s")

    @functools.partial(
        pl.kernel,
        mesh=mesh,
        out_type=jax.ShapeDtypeStruct((B,), jnp.float32),
        scratch_types=[
            pltpu.VMEM((bpw,), jnp.int32),
            pltpu.VMEM((bpw,), jnp.float32),
            pltpu.SemaphoreType.DMA,
            pltpu.SemaphoreType.DMA,
        ],
    )
    def gather_k(proj_hbm, idx_hbm, out_hbm, idx_v, val_v, sem_a, sem_b):
        wid = lax.axis_index("s") * nc + lax.axis_index("c")
        base = wid * bpw
        h = bpw // 2
        pltpu.sync_copy(idx_hbm.at[pl.ds(base, h)], idx_v.at[pl.ds(0, h)])
        c1 = pltpu.async_copy(proj_hbm.at[idx_v.at[pl.ds(0, h)]],
                              val_v.at[pl.ds(0, h)], sem_a)
        pltpu.sync_copy(idx_hbm.at[pl.ds(base + h, h)], idx_v.at[pl.ds(h, h)])
        c2 = pltpu.async_copy(proj_hbm.at[idx_v.at[pl.ds(h, h)]],
                              val_v.at[pl.ds(h, h)], sem_b)
        c1.wait()
        c2.wait()
        pltpu.sync_copy(val_v, out_hbm.at[pl.ds(base, bpw)])

    return gather_k(proj_flat, ids_flat)


def kernel(input_ids, embed_weight, value_head_weight, value_head_bias):
    B, T = input_ids.shape
    proj = _project_table_tc(embed_weight.T, value_head_weight.reshape(-1, 1),
                             value_head_bias)
    ids_flat = input_ids.T.reshape(-1).astype(jnp.int32)
    vals = _gather_scalars(proj.reshape(-1), ids_flat)
    return vals.reshape(T, B).T.reshape(B, T, 1)


# speedup vs baseline: 1.0208x; 1.0208x over previous
import functools

import jax
import jax.numpy as jnp
from jax import lax
from jax.experimental import pallas as pl
from jax.experimental.pallas import tpu as pltpu
from jax.experimental.pallas import tpu_sc as plsc

_CL = 131072


def _proj_body(w_ref, b_ref, x_ref, o_ref):
    x = x_ref[...]
    w = w_ref[...]
    p = jnp.sum(x * w, axis=0, keepdims=True) + b_ref[0, 0]
    o_ref[...] = p.reshape(o_ref.shape)


def _project_table_tc(table_t, w_col, b):
    D, V = table_t.shape
    G = (V + _CL - 1) // _CL
    return pl.pallas_call(
        _proj_body,
        grid=(G,),
        in_specs=[
            pl.BlockSpec((D, 1), lambda i: (0, 0)),
            pl.BlockSpec((1, 1), lambda i: (0, 0)),
            pl.BlockSpec((D, _CL), lambda i: (0, i)),
        ],
        out_specs=pl.BlockSpec((1, 1, _CL), lambda i: (i, 0, 0)),
        out_shape=jax.ShapeDtypeStruct((G, 1, _CL), jnp.float32),
    )(w_col, b.reshape(1, 1), table_t)


def _gather_scalars(proj_flat, ids_flat):
    info = plsc.get_sparse_core_info()
    nc, ns = info.num_cores, info.num_subcores
    nw = nc * ns
    B = ids_flat.shape[0]
    assert B % nw == 0
    bpw = B // nw

    mesh = plsc.VectorSubcoreMesh(core_axis_name="c", subcore_axis_name="s")

    @functools.partial(
        pl.kernel,
        mesh=mesh,
        out_type=jax.ShapeDtypeStruct((B,), jnp.float32),
        scratch_types=[
            pltpu.VMEM((bpw,), jnp.int32),
            pltpu.VMEM((bpw,), jnp.float32),
            pltpu.SemaphoreType.DMA,
        ],
    )
    def gather_k(proj_hbm, idx_hbm, out_hbm, idx_v, val_v, sem):
        wid = lax.axis_index("s") * nc + lax.axis_index("c")
        base = wid * bpw
        pltpu.sync_copy(idx_hbm.at[pl.ds(base, bpw)], idx_v)
        pltpu.async_copy(proj_hbm.at[idx_v], val_v, sem).wait()
        pltpu.sync_copy(val_v, out_hbm.at[pl.ds(base, bpw)])

    return gather_k(proj_flat, ids_flat)


def kernel(input_ids, embed_weight, value_head_weight, value_head_bias):
    B, T = input_ids.shape
    proj = _project_table_tc(embed_weight.T, value_head_weight.reshape(-1, 1),
                             value_head_bias)
    ids_flat = input_ids.T.reshape(-1).astype(jnp.int32)
    vals = _gather_scalars(proj.reshape(-1), ids_flat)
    return vals.reshape(T, B).T.reshape(B, T, 1)


# speedup vs baseline: 1.2121x; 1.1875x over previous
import functools

import jax
import jax.numpy as jnp
from jax import lax
from jax.experimental import pallas as pl
from jax.experimental.pallas import tpu as pltpu
from jax.experimental.pallas import tpu_sc as plsc

_CL = 131072


def _proj_body(w_ref, b_ref, x_ref, o_ref):
    x = x_ref[...]
    w = w_ref[...]
    p = jnp.sum(x * w, axis=0, keepdims=True) + b_ref[0, 0]
    o_ref[...] = p.reshape(o_ref.shape)


def _project_table_tc(table_t, w_col, b):
    D, V = table_t.shape
    G = (V + _CL - 1) // _CL
    return pl.pallas_call(
        _proj_body,
        grid=(G,),
        in_specs=[
            pl.BlockSpec((D, 1), lambda i: (0, 0)),
            pl.BlockSpec((1, 1), lambda i: (0, 0)),
            pl.BlockSpec((D, _CL), lambda i: (0, i)),
        ],
        out_specs=pl.BlockSpec((1, 1, _CL), lambda i: (i, 0, 0)),
        out_shape=jax.ShapeDtypeStruct((G, 1, _CL), jnp.float32),
    )(w_col, b.reshape(1, 1), table_t)


def _gather_scalars(proj_flat, ids_flat):
    info = plsc.get_sparse_core_info()
    nc, ns = info.num_cores, info.num_subcores
    nw = nc * ns
    B = ids_flat.shape[0]
    assert B % nw == 0
    bpw = B // nw

    mesh = plsc.VectorSubcoreMesh(core_axis_name="c", subcore_axis_name="s")

    P = proj_flat.shape[0]
    share = P // ns

    @functools.partial(
        pl.kernel,
        mesh=mesh,
        out_type=jax.ShapeDtypeStruct((B,), jnp.float32),
        scratch_types=[
            pltpu.VMEM((bpw,), jnp.int32),
            pltpu.VMEM((bpw,), jnp.float32),
            pltpu.VMEM_SHARED((P,), jnp.float32),
            pltpu.SemaphoreType.DMA,
        ],
    )
    def gather_k(proj_hbm, idx_hbm, out_hbm, idx_v, val_v, shared, sem):
        wid = lax.axis_index("s") * nc + lax.axis_index("c")
        sid = lax.axis_index("s")
        base = wid * bpw
        cp = pltpu.async_copy(proj_hbm.at[pl.ds(sid * share, share)],
                              shared.at[pl.ds(sid * share, share)], sem)
        pltpu.sync_copy(idx_hbm.at[pl.ds(base, bpw)], idx_v)
        cp.wait()
        plsc.subcore_barrier()
        pltpu.sync_copy(shared.at[idx_v], val_v)
        pltpu.sync_copy(val_v, out_hbm.at[pl.ds(base, bpw)])

    return gather_k(proj_flat, ids_flat)


def kernel(input_ids, embed_weight, value_head_weight, value_head_bias):
    B, T = input_ids.shape
    proj = _project_table_tc(embed_weight.T, value_head_weight.reshape(-1, 1),
                             value_head_bias)
    ids_flat = input_ids.T.reshape(-1).astype(jnp.int32)
    vals = _gather_scalars(proj.reshape(-1), ids_flat)
    return vals.reshape(T, B).T.reshape(B, T, 1)


# speedup vs baseline: 1.4063x; 1.1602x over previous
import functools

import jax
import jax.numpy as jnp
from jax import lax
from jax.experimental import pallas as pl
from jax.experimental.pallas import tpu as pltpu
from jax.experimental.pallas import tpu_sc as plsc

_CL = 65536


def _proj_body(w_ref, b_ref, x_ref, o_ref):
    x = x_ref[...]
    w = w_ref[...]
    p = jnp.sum(x * w, axis=0, keepdims=True) + b_ref[0, 0]
    o_ref[...] = p.reshape(o_ref.shape)


def _project_table_tc(table_t, w_col, b):
    D, V = table_t.shape
    G = (V + _CL - 1) // _CL
    return pl.pallas_call(
        _proj_body,
        grid=(G,),
        in_specs=[
            pl.BlockSpec((D, 1), lambda i: (0, 0)),
            pl.BlockSpec((1, 1), lambda i: (0, 0)),
            pl.BlockSpec((D, _CL), lambda i: (0, i)),
        ],
        out_specs=pl.BlockSpec((1, 1, _CL), lambda i: (i, 0, 0)),
        out_shape=jax.ShapeDtypeStruct((G, 1, _CL), jnp.float32),
    )(w_col, b.reshape(1, 1), table_t)


def _gather_scalars(proj_flat, ids_flat):
    info = plsc.get_sparse_core_info()
    nc, ns = info.num_cores, info.num_subcores
    nw = nc * ns
    B = ids_flat.shape[0]
    assert B % nw == 0
    bpw = B // nw

    mesh = plsc.VectorSubcoreMesh(core_axis_name="c", subcore_axis_name="s")

    P = proj_flat.shape[0]
    share = P // ns

    @functools.partial(
        pl.kernel,
        mesh=mesh,
        out_type=jax.ShapeDtypeStruct((B,), jnp.float32),
        scratch_types=[
            pltpu.VMEM((bpw,), jnp.int32),
            pltpu.VMEM((bpw,), jnp.float32),
            pltpu.VMEM_SHARED((P,), jnp.float32),
            pltpu.SemaphoreType.DMA,
        ],
    )
    def gather_k(proj_hbm, idx_hbm, out_hbm, idx_v, val_v, shared, sem):
        wid = lax.axis_index("s") * nc + lax.axis_index("c")
        sid = lax.axis_index("s")
        base = wid * bpw
        cp = pltpu.async_copy(proj_hbm.at[pl.ds(sid * share, share)],
                              shared.at[pl.ds(sid * share, share)], sem)
        pltpu.sync_copy(idx_hbm.at[pl.ds(base, bpw)], idx_v)
        cp.wait()
        plsc.subcore_barrier()
        pltpu.sync_copy(shared.at[idx_v], val_v)
        pltpu.sync_copy(val_v, out_hbm.at[pl.ds(base, bpw)])

    return gather_k(proj_flat, ids_flat)


def kernel(input_ids, embed_weight, value_head_weight, value_head_bias):
    B, T = input_ids.shape
    proj = _project_table_tc(embed_weight.T, value_head_weight.reshape(-1, 1),
                             value_head_bias)
    ids_flat = input_ids.T.reshape(-1).astype(jnp.int32)
    vals = _gather_scalars(proj.reshape(-1), ids_flat)
    return jnp.transpose(vals.reshape(T, B, 1), (1, 0, 2))
